# baseline (speedup 1.0000x reference)
import functools

import jax
import jax.numpy as jnp
from jax import lax
from jax.experimental import pallas as pl
from jax.experimental.pallas import tpu as pltpu
from jax.experimental.pallas import tpu_sc as plsc

N = 10000
E = 320000
D_IN = 128
D_HID = 64

NC = 2
NS = 16
NW = NC * NS

NP = 10240
EB = 128
EP = 327680
ROWS_W = EP // (NW * EB)
NODES_S = NP // NS
NBUF = 4
CH = 40

_mesh = plsc.VectorSubcoreMesh(core_axis_name="c", subcore_axis_name="s")


@functools.partial(
    pl.kernel,
    mesh=_mesh,
    out_type=jax.ShapeDtypeStruct((NW, NP), jnp.float32),
    compiler_params=pltpu.CompilerParams(needs_layout_passes=False, use_tc_tiling_on_sc=False),
    scratch_types=[
        pltpu.VMEM((8, EB), jnp.int32),
        pltpu.VMEM((8, EB), jnp.float32),
        pltpu.VMEM((NP,), jnp.float32),
    ],
)
def _deg_kernel(col_hbm, ew_hbm, out_hbm, col_v, ew_v, deg_v):
    cid = lax.axis_index("c")
    sid = lax.axis_index("s")
    w = cid * NS + sid

    def zero_body(i, _):
        deg_v[pl.ds(i * 16, 16)] = jnp.zeros((16,), jnp.float32)
        return 0

    lax.fori_loop(0, NP // 16, zero_body, 0)

    base = w * ROWS_W

    def step(k, _):
        pltpu.sync_copy(col_hbm.at[pl.ds(base + k * 8, 8)], col_v)
        pltpu.sync_copy(ew_hbm.at[pl.ds(base + k * 8, 8)], ew_v)
        for r in range(8):
            def inner(g, _, r=r):
                idx = col_v[r, pl.ds(g * 16, 16)]
                val = ew_v[r, pl.ds(g * 16, 16)]
                plsc.addupdate_scatter(deg_v, [idx], val)
                return 0
            lax.fori_loop(0, EB // 16, inner, 0)
        return 0

    lax.fori_loop(0, ROWS_W // 8, step, 0)
    pltpu.sync_copy(deg_v, out_hbm.at[w])


@functools.partial(
    pl.kernel,
    mesh=_mesh,
    out_type=jax.ShapeDtypeStruct((NC, NP, D_HID), jnp.float32),
    compiler_params=pltpu.CompilerParams(needs_layout_passes=False, use_tc_tiling_on_sc=False),
    scratch_types=[
        pltpu.VMEM((CH, EB), jnp.int32),
        pltpu.VMEM((CH, EB), jnp.int32),
        pltpu.VMEM((CH, EB), jnp.float32),
        [pltpu.VMEM((EB, D_HID), jnp.float32) for _ in range(NBUF)],
        pltpu.VMEM_SHARED((NP, D_HID), jnp.float32),
        pltpu.VMEM_SHARED((NP, D_HID), jnp.float32),
        pltpu.SemaphoreType.DMA((NBUF,)),
        pltpu.SemaphoreType.DMA((NBUF,)),
        pltpu.SemaphoreType.DMA,
    ],
)
def _msg_kernel(y_hbm, row_hbm, col_hbm, ew_hbm, out_hbm,
                row_v, col_v, ew_v, bufs, y_sh, acc_sh, gsem, ssem, ysem):
    cid = lax.axis_index("c")
    sid = lax.axis_index("s")
    w = cid * NS + sid
    base = w * ROWS_W
    my_nodes = pl.ds(sid * NODES_S, NODES_S)

    pltpu.async_copy(y_hbm.at[my_nodes], y_sh.at[my_nodes], ysem)

    def zrow(i, _):
        for j in range(D_HID // 16):
            bufs[0][i, pl.ds(16 * j, 16)] = jnp.zeros((16,), jnp.float32)
        return 0

    lax.fori_loop(0, EB, zrow, 0)
    for t in range(NODES_S // EB):
        pltpu.sync_copy(bufs[0],
                        acc_sh.at[pl.ds(sid * NODES_S + t * EB, EB)])

    pltpu.make_async_copy(y_hbm.at[my_nodes], y_sh.at[my_nodes], ysem).wait()
    plsc.subcore_barrier()

    def gather(k, b):
        pltpu.async_copy(y_sh.at[row_v.at[k]], bufs[b], gsem.at[b])

    def gather_wait(k, b):
        pltpu.make_async_copy(y_sh.at[row_v.at[k]], bufs[b],
                              gsem.at[b]).wait()

    def scatter(k, b):
        pltpu.async_copy(bufs[b], acc_sh.at[col_v.at[k]], ssem.at[b],
                         add=True)

    def scatter_wait(k, b):
        pltpu.make_async_copy(bufs[b], acc_sh.at[col_v.at[k]],
                              ssem.at[b]).wait()

    for h in range(ROWS_W // CH):
        hb = base + h * CH
        pltpu.sync_copy(row_hbm.at[pl.ds(hb, CH)], row_v)
        pltpu.sync_copy(col_hbm.at[pl.ds(hb, CH)], col_v)
        pltpu.sync_copy(ew_hbm.at[pl.ds(hb, CH)], ew_v)

        for b in range(NBUF - 1):
            gather(b, b)

        def superstep(s, _):
            for i in range(NBUF):
                k = s * NBUF + i
                gather_wait(k, i)

                def scale(g, _):
                    ewg = ew_v[k, pl.ds(g * 16, 16)]
                    for u in range(16):
                        sv = jnp.take_along_axis(
                            ewg, jnp.full((16,), u, jnp.int32), axis=0,
                            mode="promise_in_bounds")
                        e = g * 16 + u
                        for j in range(D_HID // 16):
                            sl = pl.ds(16 * j, 16)
                            bufs[i][e, sl] = bufs[i][e, sl] * sv
                    return 0

                lax.fori_loop(0, EB // 16, scale, 0)
                scatter(k, i)
                nb = (i + NBUF - 1) % NBUF

                @pl.when(k + NBUF - 1 < CH)
                def _():
                    @pl.when(k >= 1)
                    def _():
                        scatter_wait(k - 1, nb)
                    gather(k + NBUF - 1, nb)
            return 0

        lax.fori_loop(0, CH // NBUF, superstep, 0)
        for k in range(CH - NBUF, CH):
            scatter_wait(k, k % NBUF)
    plsc.subcore_barrier()

    for t in range(NODES_S // 64):
        o = sid * NODES_S + t * 64
        pltpu.sync_copy(acc_sh.at[pl.ds(o, 64)], out_hbm.at[cid, pl.ds(o, 64)])


def _tc_prep_body(x_ref, w_ref, degt_ref, y_ref, dinv_ref):
    deg = jnp.sum(degt_ref[...], axis=1, keepdims=True) + 1.0
    dinv = lax.rsqrt(deg)
    xw = jnp.dot(x_ref[...], w_ref[...], preferred_element_type=jnp.float32)
    y_ref[...] = xw * dinv
    dinv_ref[...] = dinv


def _tc_mid_body(a0_ref, a1_ref, y1_ref, dinv_ref, b1_ref, w2_ref, y2_ref):
    dinv = dinv_ref[...]
    h = jnp.maximum(
        (a0_ref[...] + a1_ref[...] + y1_ref[...]) * dinv + b1_ref[...], 0.0)
    y2_ref[...] = jnp.dot(
        h, w2_ref[...], preferred_element_type=jnp.float32) * dinv


def _tc_out_body(a0_ref, a1_ref, y2_ref, dinv_ref, b2_ref, out_ref):
    out_ref[...] = ((a0_ref[...] + a1_ref[...] + y2_ref[...]) * dinv_ref[...]
                    + b2_ref[...])


_tc_prep = pl.pallas_call(
    _tc_prep_body,
    out_shape=(jax.ShapeDtypeStruct((NP, D_HID), jnp.float32),
               jax.ShapeDtypeStruct((NP, 1), jnp.float32)),
)

_tc_mid = pl.pallas_call(
    _tc_mid_body,
    out_shape=jax.ShapeDtypeStruct((NP, D_HID), jnp.float32),
)

_tc_out = pl.pallas_call(
    _tc_out_body,
    out_shape=jax.ShapeDtypeStruct((NP, D_HID), jnp.float32),
)


def kernel(x, edge_index, edge_weight, W1, b1, W2, b2):
    row = edge_index[0]
    col = edge_index[1]
    pad = EP - E
    row2 = jnp.concatenate(
        [row, jnp.zeros((pad,), jnp.int32)]).reshape(EP // EB, EB)
    col2 = jnp.concatenate(
        [col, jnp.zeros((pad,), jnp.int32)]).reshape(EP // EB, EB)
    ew2 = jnp.concatenate(
        [edge_weight, jnp.zeros((pad,), jnp.float32)]).reshape(EP // EB, EB)
    x_pad = jnp.pad(x, ((0, NP - N), (0, 0)))

    degp = _deg_kernel(col2, ew2)
    y1, dinv = _tc_prep(x_pad, W1, degp.T)
    acc1 = _msg_kernel(y1, row2, col2, ew2)
    y2 = _tc_mid(acc1[0], acc1[1], y1, dinv, b1.reshape(1, D_HID), W2)
    acc2 = _msg_kernel(y2, row2, col2, ew2)
    out = _tc_out(acc2[0], acc2[1], y2, dinv, b2.reshape(1, D_HID))
    return out[:N]


# speedup vs baseline: 1.4533x; 1.4533x over previous
import functools

import jax
import jax.numpy as jnp
from jax import lax
from jax.experimental import pallas as pl
from jax.experimental.pallas import tpu as pltpu
from jax.experimental.pallas import tpu_sc as plsc

N = 10000
E = 320000
D_IN = 128
D_HID = 64

NC = 2
NS = 16
NW = NC * NS

NP = 10240
EB = 128
EP = 327680
ROWS_W = EP // (NW * EB)
NODES_S = NP // NS
NBUF = 4
CH = 40

_mesh = plsc.VectorSubcoreMesh(core_axis_name="c", subcore_axis_name="s")


@functools.partial(
    pl.kernel,
    mesh=_mesh,
    out_type=jax.ShapeDtypeStruct((NW, NP), jnp.float32),
    compiler_params=pltpu.CompilerParams(needs_layout_passes=False, use_tc_tiling_on_sc=False),
    scratch_types=[
        pltpu.VMEM((8, EB), jnp.int32),
        pltpu.VMEM((8, EB), jnp.float32),
        pltpu.VMEM((NP,), jnp.float32),
    ],
)
def _deg_kernel(col_hbm, ew_hbm, out_hbm, col_v, ew_v, deg_v):
    cid = lax.axis_index("c")
    sid = lax.axis_index("s")
    w = cid * NS + sid

    def zero_body(i, _):
        deg_v[pl.ds(i * 16, 16)] = jnp.zeros((16,), jnp.float32)
        return 0

    lax.fori_loop(0, NP // 16, zero_body, 0)

    base = w * ROWS_W

    def step(k, _):
        pltpu.sync_copy(col_hbm.at[pl.ds(base + k * 8, 8)], col_v)
        pltpu.sync_copy(ew_hbm.at[pl.ds(base + k * 8, 8)], ew_v)
        for r in range(8):
            def inner(g, _, r=r):
                idx = col_v[r, pl.ds(g * 16, 16)]
                val = ew_v[r, pl.ds(g * 16, 16)]
                plsc.addupdate_scatter(deg_v, [idx], val)
                return 0
            lax.fori_loop(0, EB // 16, inner, 0)
        return 0

    lax.fori_loop(0, ROWS_W // 8, step, 0)
    pltpu.sync_copy(deg_v, out_hbm.at[w])


@functools.partial(
    pl.kernel,
    mesh=_mesh,
    out_type=jax.ShapeDtypeStruct((NC, NP, D_HID), jnp.float32),
    compiler_params=pltpu.CompilerParams(needs_layout_passes=False, use_tc_tiling_on_sc=False),
    scratch_types=[
        pltpu.VMEM((CH, EB), jnp.int32),
        pltpu.VMEM((CH, EB), jnp.int32),
        pltpu.VMEM((CH, EB), jnp.float32),
        [pltpu.VMEM((EB, D_HID), jnp.float32) for _ in range(NBUF)],
        pltpu.VMEM_SHARED((NP, D_HID), jnp.float32),
        pltpu.VMEM_SHARED((NP, D_HID), jnp.float32),
        pltpu.SemaphoreType.DMA((NBUF,)),
        pltpu.SemaphoreType.DMA((NBUF,)),
        pltpu.SemaphoreType.DMA,
    ],
)
def _msg_kernel(y_hbm, row_hbm, col_hbm, ew_hbm, out_hbm,
                row_v, col_v, ew_v, bufs, y_sh, acc_sh, gsem, ssem, ysem):
    cid = lax.axis_index("c")
    sid = lax.axis_index("s")
    w = cid * NS + sid
    base = w * ROWS_W
    my_nodes = pl.ds(sid * NODES_S, NODES_S)

    pltpu.async_copy(y_hbm.at[my_nodes], y_sh.at[my_nodes], ysem)

    def zrow(i, _):
        for j in range(D_HID // 16):
            bufs[0][i, pl.ds(16 * j, 16)] = jnp.zeros((16,), jnp.float32)
        return 0

    lax.fori_loop(0, EB, zrow, 0)
    for t in range(NODES_S // EB):
        pltpu.sync_copy(bufs[0],
                        acc_sh.at[pl.ds(sid * NODES_S + t * EB, EB)])

    pltpu.make_async_copy(y_hbm.at[my_nodes], y_sh.at[my_nodes], ysem).wait()
    plsc.subcore_barrier()

    def gather(k, b):
        pltpu.async_copy(y_sh.at[row_v.at[k]], bufs[b], gsem.at[b])

    def gather_wait(k, b):
        pltpu.make_async_copy(y_sh.at[row_v.at[k]], bufs[b],
                              gsem.at[b]).wait()

    def scatter(k, b):
        pltpu.async_copy(bufs[b], acc_sh.at[col_v.at[k]], ssem.at[b],
                         add=True)

    def scatter_wait(k, b):
        pltpu.make_async_copy(bufs[b], acc_sh.at[col_v.at[k]],
                              ssem.at[b]).wait()

    for h in range(ROWS_W // CH):
        hb = base + h * CH
        pltpu.sync_copy(row_hbm.at[pl.ds(hb, CH)], row_v)
        pltpu.sync_copy(col_hbm.at[pl.ds(hb, CH)], col_v)
        pltpu.sync_copy(ew_hbm.at[pl.ds(hb, CH)], ew_v)

        for b in range(NBUF - 1):
            gather(b, b)

        def superstep(s, _):
            for i in range(NBUF):
                k = s * NBUF + i
                gather_wait(k, i)

                for g in range(EB // 16):
                    ewg = ew_v[k, pl.ds(g * 16, 16)]
                    for u in range(16):
                        sv = jnp.take_along_axis(
                            ewg, jnp.full((16,), u, jnp.int32), axis=0,
                            mode="promise_in_bounds")
                        e = g * 16 + u
                        for j in range(D_HID // 16):
                            sl = pl.ds(16 * j, 16)
                            bufs[i][e, sl] = bufs[i][e, sl] * sv
                scatter(k, i)
                nb = (i + NBUF - 1) % NBUF

                @pl.when(k + NBUF - 1 < CH)
                def _():
                    @pl.when(k >= 1)
                    def _():
                        scatter_wait(k - 1, nb)
                    gather(k + NBUF - 1, nb)
            return 0

        lax.fori_loop(0, CH // NBUF, superstep, 0)
        for k in range(CH - NBUF, CH):
            scatter_wait(k, k % NBUF)
    plsc.subcore_barrier()

    for t in range(NODES_S // 64):
        o = sid * NODES_S + t * 64
        pltpu.sync_copy(acc_sh.at[pl.ds(o, 64)], out_hbm.at[cid, pl.ds(o, 64)])


def _tc_prep_body(x_ref, w_ref, degt_ref, y_ref, dinv_ref):
    deg = jnp.sum(degt_ref[...], axis=1, keepdims=True) + 1.0
    dinv = lax.rsqrt(deg)
    xw = jnp.dot(x_ref[...], w_ref[...], preferred_element_type=jnp.float32)
    y_ref[...] = xw * dinv
    dinv_ref[...] = dinv


def _tc_mid_body(a0_ref, a1_ref, y1_ref, dinv_ref, b1_ref, w2_ref, y2_ref):
    dinv = dinv_ref[...]
    h = jnp.maximum(
        (a0_ref[...] + a1_ref[...] + y1_ref[...]) * dinv + b1_ref[...], 0.0)
    y2_ref[...] = jnp.dot(
        h, w2_ref[...], preferred_element_type=jnp.float32) * dinv


def _tc_out_body(a0_ref, a1_ref, y2_ref, dinv_ref, b2_ref, out_ref):
    out_ref[...] = ((a0_ref[...] + a1_ref[...] + y2_ref[...]) * dinv_ref[...]
                    + b2_ref[...])


_tc_prep = pl.pallas_call(
    _tc_prep_body,
    out_shape=(jax.ShapeDtypeStruct((NP, D_HID), jnp.float32),
               jax.ShapeDtypeStruct((NP, 1), jnp.float32)),
)

_tc_mid = pl.pallas_call(
    _tc_mid_body,
    out_shape=jax.ShapeDtypeStruct((NP, D_HID), jnp.float32),
)

_tc_out = pl.pallas_call(
    _tc_out_body,
    out_shape=jax.ShapeDtypeStruct((NP, D_HID), jnp.float32),
)


def kernel(x, edge_index, edge_weight, W1, b1, W2, b2):
    row = edge_index[0]
    col = edge_index[1]
    pad = EP - E
    row2 = jnp.concatenate(
        [row, jnp.zeros((pad,), jnp.int32)]).reshape(EP // EB, EB)
    col2 = jnp.concatenate(
        [col, jnp.zeros((pad,), jnp.int32)]).reshape(EP // EB, EB)
    ew2 = jnp.concatenate(
        [edge_weight, jnp.zeros((pad,), jnp.float32)]).reshape(EP // EB, EB)
    x_pad = jnp.pad(x, ((0, NP - N), (0, 0)))

    degp = _deg_kernel(col2, ew2)
    y1, dinv = _tc_prep(x_pad, W1, degp.T)
    acc1 = _msg_kernel(y1, row2, col2, ew2)
    y2 = _tc_mid(acc1[0], acc1[1], y1, dinv, b1.reshape(1, D_HID), W2)
    acc2 = _msg_kernel(y2, row2, col2, ew2)
    out = _tc_out(acc2[0], acc2[1], y2, dinv, b2.reshape(1, D_HID))
    return out[:N]


# speedup vs baseline: 1.5171x; 1.0439x over previous
import functools

import jax
import jax.numpy as jnp
from jax import lax
from jax.experimental import pallas as pl
from jax.experimental.pallas import tpu as pltpu
from jax.experimental.pallas import tpu_sc as plsc

N = 10000
E = 320000
D_IN = 128
D_HID = 64

NC = 2
NS = 16
NW = NC * NS

NP = 10240
EB = 128
EP = 327680
ROWS_W = EP // (NW * EB)
NODES_S = NP // NS
NBUF = 4
CH = 40

_mesh = plsc.VectorSubcoreMesh(core_axis_name="c", subcore_axis_name="s")


@functools.partial(
    pl.kernel,
    mesh=_mesh,
    out_type=jax.ShapeDtypeStruct((NW, NP), jnp.float32),
    compiler_params=pltpu.CompilerParams(needs_layout_passes=False, use_tc_tiling_on_sc=False),
    scratch_types=[
        pltpu.VMEM((8, EB), jnp.int32),
        pltpu.VMEM((8, EB), jnp.float32),
        pltpu.VMEM((NP,), jnp.float32),
    ],
)
def _deg_kernel(col_hbm, ew_hbm, out_hbm, col_v, ew_v, deg_v):
    cid = lax.axis_index("c")
    sid = lax.axis_index("s")
    w = cid * NS + sid

    def zero_body(i, _):
        deg_v[pl.ds(i * 16, 16)] = jnp.zeros((16,), jnp.float32)
        return 0

    lax.fori_loop(0, NP // 16, zero_body, 0)

    base = w * ROWS_W

    def step(k, _):
        pltpu.sync_copy(col_hbm.at[pl.ds(base + k * 8, 8)], col_v)
        pltpu.sync_copy(ew_hbm.at[pl.ds(base + k * 8, 8)], ew_v)
        for r in range(8):
            def inner(g, _, r=r):
                idx = col_v[r, pl.ds(g * 16, 16)]
                val = ew_v[r, pl.ds(g * 16, 16)]
                plsc.addupdate_scatter(deg_v, [idx], val)
                return 0
            lax.fori_loop(0, EB // 16, inner, 0)
        return 0

    lax.fori_loop(0, ROWS_W // 8, step, 0)
    pltpu.sync_copy(deg_v, out_hbm.at[w])


@functools.partial(
    pl.kernel,
    mesh=_mesh,
    out_type=jax.ShapeDtypeStruct((NC, NP, D_HID), jnp.float32),
    compiler_params=pltpu.CompilerParams(needs_layout_passes=False, use_tc_tiling_on_sc=False),
    scratch_types=[
        pltpu.VMEM((CH, EB), jnp.int32),
        pltpu.VMEM((CH, EB), jnp.int32),
        pltpu.VMEM((CH, EB), jnp.float32),
        [pltpu.VMEM((EB, D_HID), jnp.float32) for _ in range(NBUF)],
        pltpu.VMEM_SHARED((NP, D_HID), jnp.float32),
        pltpu.VMEM_SHARED((NP, D_HID), jnp.float32),
        pltpu.SemaphoreType.DMA((NBUF,)),
        pltpu.SemaphoreType.DMA((NBUF,)),
        pltpu.SemaphoreType.DMA,
    ],
)
def _msg_kernel(y_hbm, row_hbm, col_hbm, ew_hbm, out_hbm,
                row_v, col_v, ew_v, bufs, y_sh, acc_sh, gsem, ssem, ysem):
    cid = lax.axis_index("c")
    sid = lax.axis_index("s")
    w = cid * NS + sid
    base = w * ROWS_W
    my_nodes = pl.ds(sid * NODES_S, NODES_S)

    pltpu.async_copy(y_hbm.at[my_nodes], y_sh.at[my_nodes], ysem)

    def zrow(i, _):
        for j in range(D_HID // 16):
            bufs[0][i, pl.ds(16 * j, 16)] = jnp.zeros((16,), jnp.float32)
        return 0

    lax.fori_loop(0, EB, zrow, 0)
    for t in range(NODES_S // EB):
        pltpu.sync_copy(bufs[0],
                        acc_sh.at[pl.ds(sid * NODES_S + t * EB, EB)])

    pltpu.make_async_copy(y_hbm.at[my_nodes], y_sh.at[my_nodes], ysem).wait()
    plsc.subcore_barrier()

    def gather(k, b):
        pltpu.async_copy(y_sh.at[row_v.at[k]], bufs[b], gsem.at[b])

    def gather_wait(k, b):
        pltpu.make_async_copy(y_sh.at[row_v.at[k]], bufs[b],
                              gsem.at[b]).wait()

    def scatter(k, b):
        pltpu.async_copy(bufs[b], acc_sh.at[col_v.at[k]], ssem.at[b],
                         add=True)

    def scatter_wait(k, b):
        pltpu.make_async_copy(bufs[b], acc_sh.at[col_v.at[k]],
                              ssem.at[b]).wait()

    for h in range(ROWS_W // CH):
        hb = base + h * CH
        pltpu.sync_copy(row_hbm.at[pl.ds(hb, CH)], row_v)
        pltpu.sync_copy(col_hbm.at[pl.ds(hb, CH)], col_v)
        pltpu.sync_copy(ew_hbm.at[pl.ds(hb, CH)], ew_v)

        for b in range(NBUF - 1):
            gather(b, b)

        def superstep(s, _):
            for i in range(NBUF):
                k = s * NBUF + i
                gather_wait(k, i)

                for g in range(EB // 16):
                    ewg = ew_v[k, pl.ds(g * 16, 16)]
                    for u in range(16):
                        sv = jnp.take_along_axis(
                            ewg, jnp.full((16,), u, jnp.int32), axis=0,
                            mode="promise_in_bounds")
                        e = g * 16 + u
                        for j in range(D_HID // 16):
                            sl = pl.ds(16 * j, 16)
                            bufs[i][e, sl] = bufs[i][e, sl] * sv
                scatter(k, i)
                nb = (i + NBUF - 1) % NBUF

                @pl.when(k + NBUF - 1 < CH)
                def _():
                    @pl.when(k >= 1)
                    def _():
                        scatter_wait(k - 1, nb)
                    gather(k + NBUF - 1, nb)
            return 0

        lax.fori_loop(0, CH // NBUF, superstep, 0)
        for k in range(CH - NBUF, CH):
            scatter_wait(k, k % NBUF)
    plsc.subcore_barrier()

    for t in range(NODES_S // 64):
        o = sid * NODES_S + t * 64
        pltpu.sync_copy(acc_sh.at[pl.ds(o, 64)], out_hbm.at[cid, pl.ds(o, 64)])


def _tc_mm_body(x_ref, w_ref, xw_ref):
    xw_ref[...] = jnp.dot(
        x_ref[...], w_ref[...], preferred_element_type=jnp.float32)


def _tc_norm_body(xw_ref, degt_ref, y_ref, dinv_ref):
    deg = jnp.sum(degt_ref[...], axis=1, keepdims=True) + 1.0
    dinv = lax.rsqrt(deg)
    y_ref[...] = xw_ref[...] * dinv
    dinv_ref[...] = dinv


def _tc_mid_body(a_ref, y1_ref, dinv_ref, b1_ref, w2_ref, y2_ref):
    dinv = dinv_ref[...]
    h = jnp.maximum(
        (a_ref[0] + a_ref[1] + y1_ref[...]) * dinv + b1_ref[...], 0.0)
    y2_ref[...] = jnp.dot(
        h, w2_ref[...], preferred_element_type=jnp.float32) * dinv


def _tc_out_body(a_ref, y2_ref, dinv_ref, b2_ref, out_ref):
    out_ref[...] = ((a_ref[0, :N] + a_ref[1, :N] + y2_ref[:N])
                    * dinv_ref[:N] + b2_ref[...])


_tc_mm = pl.pallas_call(
    _tc_mm_body,
    out_shape=jax.ShapeDtypeStruct((NP, D_HID), jnp.float32),
)

_tc_norm = pl.pallas_call(
    _tc_norm_body,
    out_shape=(jax.ShapeDtypeStruct((NP, D_HID), jnp.float32),
               jax.ShapeDtypeStruct((NP, 1), jnp.float32)),
)

_tc_mid = pl.pallas_call(
    _tc_mid_body,
    out_shape=jax.ShapeDtypeStruct((NP, D_HID), jnp.float32),
)

_tc_out = pl.pallas_call(
    _tc_out_body,
    out_shape=jax.ShapeDtypeStruct((N, D_HID), jnp.float32),
)


def kernel(x, edge_index, edge_weight, W1, b1, W2, b2):
    row = edge_index[0]
    col = edge_index[1]
    pad = EP - E
    row2 = jnp.concatenate(
        [row, jnp.zeros((pad,), jnp.int32)]).reshape(EP // EB, EB)
    col2 = jnp.concatenate(
        [col, jnp.zeros((pad,), jnp.int32)]).reshape(EP // EB, EB)
    ew2 = jnp.concatenate(
        [edge_weight, jnp.zeros((pad,), jnp.float32)]).reshape(EP // EB, EB)
    x_pad = jnp.pad(x, ((0, NP - N), (0, 0)))

    xw = _tc_mm(x_pad, W1)
    degp = _deg_kernel(col2, ew2)
    y1, dinv = _tc_norm(xw, degp.T)
    acc1 = _msg_kernel(y1, row2, col2, ew2)
    y2 = _tc_mid(acc1, y1, dinv, b1.reshape(1, D_HID), W2)
    acc2 = _msg_kernel(y2, row2, col2, ew2)
    out = _tc_out(acc2, y2, dinv, b2.reshape(1, D_HID))
    return out


# speedup vs baseline: 1.5734x; 1.0371x over previous
import functools

import jax
import jax.numpy as jnp
from jax import lax
from jax.experimental import pallas as pl
from jax.experimental.pallas import tpu as pltpu
from jax.experimental.pallas import tpu_sc as plsc

N = 10000
E = 320000
D_IN = 128
D_HID = 64

NC = 2
NS = 16
NW = NC * NS

NP = 10240
EB = 128
EP = 327680
ROWS_W = EP // (NW * EB)
NODES_S = NP // NS
NBUF = 4
CH = 40

_mesh = plsc.VectorSubcoreMesh(core_axis_name="c", subcore_axis_name="s")


@functools.partial(
    pl.kernel,
    mesh=_mesh,
    out_type=jax.ShapeDtypeStruct((NW, NP), jnp.float32),
    compiler_params=pltpu.CompilerParams(needs_layout_passes=False, use_tc_tiling_on_sc=False),
    scratch_types=[
        pltpu.VMEM((ROWS_W, EB), jnp.int32),
        pltpu.VMEM((ROWS_W, EB), jnp.float32),
        pltpu.VMEM((NP,), jnp.float32),
        pltpu.SemaphoreType.DMA,
        pltpu.SemaphoreType.DMA,
    ],
)
def _deg_kernel(col_hbm, ew_hbm, out_hbm, col_v, ew_v, deg_v, csem, esem):
    cid = lax.axis_index("c")
    sid = lax.axis_index("s")
    w = cid * NS + sid
    base = w * ROWS_W

    pltpu.async_copy(col_hbm.at[pl.ds(base, ROWS_W)], col_v, csem)
    pltpu.async_copy(ew_hbm.at[pl.ds(base, ROWS_W)], ew_v, esem)

    def zero_body(i, _):
        deg_v[pl.ds(i * 16, 16)] = jnp.zeros((16,), jnp.float32)
        return 0

    lax.fori_loop(0, NP // 16, zero_body, 0)
    pltpu.make_async_copy(col_hbm.at[pl.ds(base, ROWS_W)], col_v, csem).wait()
    pltpu.make_async_copy(ew_hbm.at[pl.ds(base, ROWS_W)], ew_v, esem).wait()

    def step(k, _):
        for g in range(EB // 16):
            idx = col_v[k, pl.ds(g * 16, 16)]
            val = ew_v[k, pl.ds(g * 16, 16)]
            plsc.addupdate_scatter(deg_v, [idx], val)
        return 0

    lax.fori_loop(0, ROWS_W, step, 0)
    pltpu.sync_copy(deg_v, out_hbm.at[w])


@functools.partial(
    pl.kernel,
    mesh=_mesh,
    out_type=jax.ShapeDtypeStruct((NC, NP, D_HID), jnp.float32),
    compiler_params=pltpu.CompilerParams(needs_layout_passes=False, use_tc_tiling_on_sc=False),
    scratch_types=[
        pltpu.VMEM((CH, EB), jnp.int32),
        pltpu.VMEM((CH, EB), jnp.int32),
        pltpu.VMEM((CH, EB), jnp.float32),
        [pltpu.VMEM((EB, D_HID), jnp.float32) for _ in range(NBUF)],
        pltpu.VMEM_SHARED((NP, D_HID), jnp.float32),
        pltpu.VMEM_SHARED((NP, D_HID), jnp.float32),
        pltpu.SemaphoreType.DMA((NBUF,)),
        pltpu.SemaphoreType.DMA((NBUF,)),
        pltpu.SemaphoreType.DMA,
    ],
)
def _msg_kernel(y_hbm, row_hbm, col_hbm, ew_hbm, out_hbm,
                row_v, col_v, ew_v, bufs, y_sh, acc_sh, gsem, ssem, ysem):
    cid = lax.axis_index("c")
    sid = lax.axis_index("s")
    w = cid * NS + sid
    base = w * ROWS_W
    my_nodes = pl.ds(sid * NODES_S, NODES_S)

    pltpu.async_copy(y_hbm.at[my_nodes], y_sh.at[my_nodes], ysem)

    def zrow(i, _):
        for j in range(D_HID // 16):
            bufs[0][i, pl.ds(16 * j, 16)] = jnp.zeros((16,), jnp.float32)
        return 0

    lax.fori_loop(0, EB, zrow, 0)
    for t in range(NODES_S // EB):
        pltpu.sync_copy(bufs[0],
                        acc_sh.at[pl.ds(sid * NODES_S + t * EB, EB)])

    pltpu.make_async_copy(y_hbm.at[my_nodes], y_sh.at[my_nodes], ysem).wait()
    plsc.subcore_barrier()

    def gather(k, b):
        pltpu.async_copy(y_sh.at[row_v.at[k]], bufs[b], gsem.at[b])

    def gather_wait(k, b):
        pltpu.make_async_copy(y_sh.at[row_v.at[k]], bufs[b],
                              gsem.at[b]).wait()

    def scatter(k, b):
        pltpu.async_copy(bufs[b], acc_sh.at[col_v.at[k]], ssem.at[b],
                         add=True)

    def scatter_wait(k, b):
        pltpu.make_async_copy(bufs[b], acc_sh.at[col_v.at[k]],
                              ssem.at[b]).wait()

    for h in range(ROWS_W // CH):
        hb = base + h * CH
        pltpu.sync_copy(row_hbm.at[pl.ds(hb, CH)], row_v)
        pltpu.sync_copy(col_hbm.at[pl.ds(hb, CH)], col_v)
        pltpu.sync_copy(ew_hbm.at[pl.ds(hb, CH)], ew_v)

        for b in range(NBUF - 1):
            gather(b, b)

        def superstep(s, _):
            for i in range(NBUF):
                k = s * NBUF + i
                gather_wait(k, i)

                for g in range(EB // 16):
                    ewg = ew_v[k, pl.ds(g * 16, 16)]
                    for u in range(16):
                        sv = jnp.take_along_axis(
                            ewg, jnp.full((16,), u, jnp.int32), axis=0,
                            mode="promise_in_bounds")
                        e = g * 16 + u
                        for j in range(D_HID // 16):
                            sl = pl.ds(16 * j, 16)
                            bufs[i][e, sl] = bufs[i][e, sl] * sv
                scatter(k, i)
                nb = (i + NBUF - 1) % NBUF

                @pl.when(k + NBUF - 1 < CH)
                def _():
                    @pl.when(k >= 1)
                    def _():
                        scatter_wait(k - 1, nb)
                    gather(k + NBUF - 1, nb)
            return 0

        lax.fori_loop(0, CH // NBUF, superstep, 0)
        for k in range(CH - NBUF, CH):
            scatter_wait(k, k % NBUF)
    plsc.subcore_barrier()

    for t in range(NODES_S // 64):
        o = sid * NODES_S + t * 64
        pltpu.sync_copy(acc_sh.at[pl.ds(o, 64)], out_hbm.at[cid, pl.ds(o, 64)])


def _tc_mm_body(x_ref, w_ref, xw_ref):
    xw_ref[...] = jnp.dot(
        x_ref[...], w_ref[...], preferred_element_type=jnp.float32)


def _tc_norm_body(xw_ref, degt_ref, y_ref, dinv_ref):
    deg = jnp.sum(degt_ref[...], axis=1, keepdims=True) + 1.0
    dinv = lax.rsqrt(deg)
    y_ref[...] = xw_ref[...] * dinv
    dinv_ref[...] = dinv


def _tc_mid_body(a_ref, y1_ref, dinv_ref, b1_ref, w2_ref, y2_ref):
    dinv = dinv_ref[...]
    h = jnp.maximum(
        (a_ref[0] + a_ref[1] + y1_ref[...]) * dinv + b1_ref[...], 0.0)
    y2_ref[...] = jnp.dot(
        h, w2_ref[...], preferred_element_type=jnp.float32) * dinv


def _tc_out_body(a_ref, y2_ref, dinv_ref, b2_ref, out_ref):
    out_ref[...] = ((a_ref[0, :N] + a_ref[1, :N] + y2_ref[:N])
                    * dinv_ref[:N] + b2_ref[...])


_tc_mm = pl.pallas_call(
    _tc_mm_body,
    out_shape=jax.ShapeDtypeStruct((NP, D_HID), jnp.float32),
)

_tc_norm = pl.pallas_call(
    _tc_norm_body,
    out_shape=(jax.ShapeDtypeStruct((NP, D_HID), jnp.float32),
               jax.ShapeDtypeStruct((NP, 1), jnp.float32)),
)

_tc_mid = pl.pallas_call(
    _tc_mid_body,
    out_shape=jax.ShapeDtypeStruct((NP, D_HID), jnp.float32),
)

_tc_out = pl.pallas_call(
    _tc_out_body,
    out_shape=jax.ShapeDtypeStruct((N, D_HID), jnp.float32),
)


def kernel(x, edge_index, edge_weight, W1, b1, W2, b2):
    row = edge_index[0]
    col = edge_index[1]
    pad = EP - E
    row2 = jnp.concatenate(
        [row, jnp.zeros((pad,), jnp.int32)]).reshape(EP // EB, EB)
    col2 = jnp.concatenate(
        [col, jnp.zeros((pad,), jnp.int32)]).reshape(EP // EB, EB)
    ew2 = jnp.concatenate(
        [edge_weight, jnp.zeros((pad,), jnp.float32)]).reshape(EP // EB, EB)
    x_pad = jnp.pad(x, ((0, NP - N), (0, 0)))

    xw = _tc_mm(x_pad, W1)
    degp = _deg_kernel(col2, ew2)
    y1, dinv = _tc_norm(xw, degp.T)
    acc1 = _msg_kernel(y1, row2, col2, ew2)
    y2 = _tc_mid(acc1, y1, dinv, b1.reshape(1, D_HID), W2)
    acc2 = _msg_kernel(y2, row2, col2, ew2)
    out = _tc_out(acc2, y2, dinv, b2.reshape(1, D_HID))
    return out
